# baseline (speedup 1.0000x reference)
import functools

import jax
import jax.numpy as jnp
from jax import lax
from jax.experimental import pallas as pl
from jax.experimental.pallas import tpu as pltpu
from jax.experimental.pallas import tpu_sc as plsc

_N = 16384
_C = 1000
_K = 4096
_INT_MIN = -2147483648

_NW = 32
_ROWS_W = _N // _NW
_CH = 16
_NCH = _ROWS_W // _CH
_CHW = _CH * _C
_FULL = _C // 16
_TAIL = _C - _FULL * 16


def _row_sums(buf, tgtb, sumsb, xtb, chunk):
    lanes = lax.iota(jnp.int32, 16)
    tailmask = lanes < _TAIL

    def row_body(rr, res):
        base = rr * _C
        acc = jnp.exp(buf[pl.ds(base, 16)])
        for c in range(1, _FULL):
            acc = acc + jnp.exp(buf[pl.ds(base + c * 16, 16)])
        tail = buf[pl.ds(base + _FULL * 16, 16)]
        acc = acc + jnp.where(tailmask, jnp.exp(tail), jnp.float32(0.0))
        s = jnp.sum(acc)
        return jnp.where(lanes == rr, s, res)

    res = lax.fori_loop(0, _CH, row_body, jnp.zeros((16,), jnp.float32))
    sumsb[pl.ds(chunk * _CH, 16)] = res
    t_vec = tgtb[pl.ds(chunk * _CH, 16)]
    addr = lanes * _C + t_vec
    xtb[pl.ds(chunk * _CH, 16)] = plsc.load_gather(buf, [addr])


def _sc_ce(x_hbm, tgt_hbm, sums_out, xt_out,
           buf0, buf1, sumsb, xtb, tgtb, sem0, sem1):
    wid = lax.axis_index("s") * 2 + lax.axis_index("c")
    row0 = wid * _ROWS_W
    base = row0 * _C

    pltpu.sync_copy(tgt_hbm.at[pl.ds(row0, _ROWS_W)], tgtb)

    def chunk_src(g):
        gg = jnp.minimum(g, _NCH - 1)
        return x_hbm.at[pl.ds(base + gg * _CHW, _CHW)]

    pltpu.async_copy(chunk_src(jnp.int32(0)), buf0.at[pl.ds(0, _CHW)], sem0)
    pltpu.async_copy(chunk_src(jnp.int32(1)), buf1.at[pl.ds(0, _CHW)], sem1)

    def outer(g, carry):
        g0 = g * 2
        pltpu.make_async_copy(chunk_src(g0), buf0.at[pl.ds(0, _CHW)], sem0).wait()
        _row_sums(buf0, tgtb, sumsb, xtb, g0)
        pltpu.async_copy(chunk_src(g0 + 2), buf0.at[pl.ds(0, _CHW)], sem0)

        g1 = g0 + 1
        pltpu.make_async_copy(chunk_src(g1), buf1.at[pl.ds(0, _CHW)], sem1).wait()
        _row_sums(buf1, tgtb, sumsb, xtb, g1)
        pltpu.async_copy(chunk_src(g1 + 2), buf1.at[pl.ds(0, _CHW)], sem1)
        return carry

    lax.fori_loop(0, _NCH // 2, outer, jnp.int32(0))

    pltpu.make_async_copy(chunk_src(jnp.int32(_NCH - 1)),
                          buf0.at[pl.ds(0, _CHW)], sem0).wait()
    pltpu.make_async_copy(chunk_src(jnp.int32(_NCH - 1)),
                          buf1.at[pl.ds(0, _CHW)], sem1).wait()

    pltpu.sync_copy(sumsb, sums_out.at[pl.ds(row0, _ROWS_W)])
    pltpu.sync_copy(xtb, xt_out.at[pl.ds(row0, _ROWS_W)])


def _select_body(s_ref, xt_ref, out_ref):
    L = jnp.log(s_ref[...]) - xt_ref[...]
    b = lax.bitcast_convert_type(L, jnp.int32)
    keys = jnp.where(b < 0, jnp.bitwise_not(b) ^ jnp.int32(_INT_MIN), b)

    def step(i, tu):
        bit = jnp.int32(31) - i
        cand = tu | (jnp.int32(1) << bit)
        cand_s = cand ^ jnp.int32(_INT_MIN)
        cnt = jnp.sum((keys >= cand_s).astype(jnp.int32))
        return jnp.where(cnt >= _K, cand, tu)

    tu = lax.fori_loop(0, 32, step, jnp.int32(0))
    tu_s = tu ^ jnp.int32(_INT_MIN)
    tb = jnp.where(tu < 0, tu ^ jnp.int32(_INT_MIN), jnp.bitwise_not(tu))
    tval = lax.bitcast_convert_type(tb, jnp.float32)

    gt = keys > tu_s
    cnt_gt = jnp.sum(gt.astype(jnp.float32))
    sum_gt = jnp.sum(jnp.where(gt, L, 0.0))
    res = (sum_gt + (jnp.float32(_K) - cnt_gt) * tval) / _K
    out_ref[...] = res.reshape(1, 1)


@functools.partial(jax.jit)
def kernel(inputs, targets):
    mesh = plsc.VectorSubcoreMesh(core_axis_name="c", subcore_axis_name="s")
    sc_ce = functools.partial(
        pl.kernel,
        mesh=mesh,
        compiler_params=pltpu.CompilerParams(needs_layout_passes=False),
        out_type=[
            jax.ShapeDtypeStruct((_N,), jnp.float32),
            jax.ShapeDtypeStruct((_N,), jnp.float32),
        ],
        scratch_types=[
            pltpu.VMEM((_CHW + 16,), jnp.float32),
            pltpu.VMEM((_CHW + 16,), jnp.float32),
            pltpu.VMEM((_ROWS_W,), jnp.float32),
            pltpu.VMEM((_ROWS_W,), jnp.float32),
            pltpu.VMEM((_ROWS_W,), jnp.int32),
            pltpu.SemaphoreType.DMA,
            pltpu.SemaphoreType.DMA,
        ],
    )(_sc_ce)
    sums, xt = sc_ce(inputs.reshape(_N * _C), targets)

    out = pl.pallas_call(
        _select_body,
        in_specs=[
            pl.BlockSpec((128, 128), lambda: (0, 0)),
            pl.BlockSpec((128, 128), lambda: (0, 0)),
        ],
        out_specs=pl.BlockSpec((1, 1), lambda: (0, 0)),
        out_shape=jax.ShapeDtypeStruct((1, 1), jnp.float32),
    )(sums.reshape(128, 128), xt.reshape(128, 128))
    return out[0, 0]
